# Initial kernel scaffold; baseline (speedup 1.0000x reference)
import jax
import jax.numpy as jnp
from jax.experimental import pallas as pl


def kernel(x, edge_index, W1, b1, W2, b2, W3, b3, W4, b4, W5, b5):
    raise NotImplementedError("write your pallas kernel here")


# speedup vs baseline: 14.5894x; 14.5894x over previous
import functools

import jax
import jax.numpy as jnp
from jax import lax
from jax.experimental import pallas as pl
from jax.experimental.pallas import tpu as pltpu
from jax.experimental.pallas import tpu_sc as plsc

NC = 2
NS = 16


def _mesh():
    return plsc.VectorSubcoreMesh(
        core_axis_name="c", subcore_axis_name="s", num_cores=NC, num_subcores=NS
    )


@functools.lru_cache(maxsize=None)
def _make_deg(N, E):
    NW = NC * NS
    assert E % NW == 0
    epw = E // NW
    CH = 80
    assert epw % CH == 0
    nch = epw // CH

    @functools.partial(
        pl.kernel,
        out_type=jax.ShapeDtypeStruct((NC * N,), jnp.float32),
        mesh=_mesh(),
        scratch_types=[
            pltpu.VMEM((nch, CH), jnp.int32),
            pltpu.VMEM((CH,), jnp.float32),
            pltpu.VMEM((640,), jnp.float32),
            pltpu.VMEM_SHARED((N,), jnp.float32),
        ],
    )
    def deg_kernel(dst_hbm, out_hbm, dst_v, ones, zeros, acc):
        cid = lax.axis_index("c")
        sid = lax.axis_index("s")
        wid = sid * NC + cid
        pltpu.sync_copy(dst_hbm.at[wid], dst_v)

        def init16(i, _):
            ones[pl.ds(i * 16, 16)] = jnp.ones((16,), jnp.float32)
            zeros[pl.ds(i * 16, 16)] = jnp.zeros((16,), jnp.float32)
            return 0

        lax.fori_loop(0, CH // 16, init16, 0, unroll=True)

        def zrest(i, _):
            zeros[pl.ds(CH + i * 16, 16)] = jnp.zeros((16,), jnp.float32)
            return 0

        lax.fori_loop(0, (640 - CH) // 16, zrest, 0, unroll=True)

        my0 = pl.multiple_of(sid * 640, 8)
        @pl.when(sid < NS - 1)
        def _():
            pltpu.sync_copy(zeros, acc.at[pl.ds(my0, 640)])

        @pl.when(sid == NS - 1)
        def _():
            rem = N - 640 * (NS - 1)
            for k in range(0, rem, 80):
                pltpu.sync_copy(zeros.at[pl.ds(0, 80)], acc.at[pl.ds(640 * (NS - 1) + k, 80)])

        plsc.subcore_barrier()

        def body(c, _):
            pltpu.sync_copy(ones, acc.at[dst_v.at[c]], add=True)
            return 0

        lax.fori_loop(0, nch, body, 0)
        plsc.subcore_barrier()

        cbase = cid * N
        @pl.when(sid < NS - 1)
        def _():
            pltpu.sync_copy(acc.at[pl.ds(my0, 640)], zeros)
            pltpu.sync_copy(zeros, out_hbm.at[pl.ds(pl.multiple_of(cbase + my0, 8), 640)])

        @pl.when(sid == NS - 1)
        def _():
            rem = N - 640 * (NS - 1)
            for k in range(0, rem, 80):
                o = 640 * (NS - 1) + k
                pltpu.sync_copy(acc.at[pl.ds(o, 80)], zeros.at[pl.ds(0, 80)])
                pltpu.sync_copy(zeros.at[pl.ds(0, 80)],
                                out_hbm.at[pl.ds(pl.multiple_of(cbase + o, 8), 80)])

    return deg_kernel


@functools.lru_cache(maxsize=None)
def _make_scatter(N, E, F):
    NW = NC * NS
    assert E % NW == 0
    epw = E // NW
    CH = 80
    assert epw % CH == 0
    nch = epw // CH
    NB = 5
    assert nch % NB == 0
    bch = nch // NB
    ZR = 80
    FULL = 640
    REM = N - FULL * (NS - 1)
    assert 0 < REM <= FULL and REM % ZR == 0

    @functools.partial(
        pl.kernel,
        out_type=jax.ShapeDtypeStruct((NC, N, F), jnp.float32),
        mesh=_mesh(),
        scratch_types=[
            pltpu.VMEM((bch, CH), jnp.int32),
            pltpu.VMEM((bch, CH), jnp.int32),
            pltpu.VMEM((CH, F), jnp.float32),
            pltpu.VMEM_SHARED((N, F), jnp.float32),
            pltpu.SemaphoreType.DMA,
        ],
    )
    def scat_kernel(src_hbm, dst_hbm, tbl_hbm, out_hbm,
                    src_v, dst_v, rows, acc, sem):
        cid = lax.axis_index("c")
        sid = lax.axis_index("s")
        wid = sid * NC + cid

        def zrow(r, _):
            for j in range(F // 16):
                rows[r, pl.ds(j * 16, 16)] = jnp.zeros((16,), jnp.float32)
            return 0

        lax.fori_loop(0, ZR, zrow, 0)

        my0 = sid * FULL

        @pl.when(sid < NS - 1)
        def _():
            for k in range(0, FULL, ZR):
                o = pl.multiple_of(my0 + k, 8)
                pltpu.sync_copy(rows, acc.at[pl.ds(o, ZR)])

        @pl.when(sid == NS - 1)
        def _():
            for k in range(0, REM, ZR):
                pltpu.sync_copy(rows, acc.at[pl.ds(FULL * (NS - 1) + k, ZR)])

        plsc.subcore_barrier()

        def batch(b, _):
            pltpu.sync_copy(src_hbm.at[wid, b], src_v)
            pltpu.sync_copy(dst_hbm.at[wid, b], dst_v)

            def body(c, _):
                pltpu.async_copy(tbl_hbm.at[src_v.at[c]], rows, sem).wait()
                pltpu.sync_copy(rows, acc.at[dst_v.at[c]], add=True)
                return 0

            lax.fori_loop(0, bch, body, 0)
            return 0

        lax.fori_loop(0, NB, batch, 0)
        plsc.subcore_barrier()

        @pl.when(sid < NS - 1)
        def _():
            for k in range(0, FULL, ZR):
                o = pl.multiple_of(my0 + k, 8)
                pltpu.sync_copy(acc.at[pl.ds(o, ZR)], rows)
                pltpu.sync_copy(rows, out_hbm.at[cid, pl.ds(o, ZR)])

        @pl.when(sid == NS - 1)
        def _():
            for k in range(0, REM, ZR):
                o = FULL * (NS - 1) + k
                pltpu.sync_copy(acc.at[pl.ds(o, ZR)], rows)
                pltpu.sync_copy(rows, out_hbm.at[cid, pl.ds(o, ZR)])

    return scat_kernel


_R = 1000


def _row_grid(N):
    assert N % _R == 0
    return N // _R


@functools.lru_cache(maxsize=None)
def _make_prescale(N, D):
    def body(deg0_ref, deg1_ref, x_ref, dinv_ref, xs_ref):
        deg = deg0_ref[...] + deg1_ref[...] + 1.0
        dinv = lax.rsqrt(jnp.maximum(deg, 1e-12))
        dinv_ref[...] = dinv
        xs_ref[...] = x_ref[...] * dinv

    return pl.pallas_call(
        body,
        grid=(_row_grid(N),),
        in_specs=[
            pl.BlockSpec((_R, 1), lambda i: (i, 0)),
            pl.BlockSpec((_R, 1), lambda i: (i, 0)),
            pl.BlockSpec((_R, D), lambda i: (i, 0)),
        ],
        out_specs=[
            pl.BlockSpec((_R, 1), lambda i: (i, 0)),
            pl.BlockSpec((_R, D), lambda i: (i, 0)),
        ],
        out_shape=[
            jax.ShapeDtypeStruct((N, 1), jnp.float32),
            jax.ShapeDtypeStruct((N, D), jnp.float32),
        ],
    )


@functools.lru_cache(maxsize=None)
def _make_stage_a(N, D, H):
    def body(u_ref, xs_ref, dinv_ref, w1_ref, b1_ref, w2_ref, out_ref):
        dinv = dinv_ref[...]
        t = dinv * (u_ref[0] + u_ref[1] + xs_ref[...])
        z1 = jnp.maximum(
            jax.lax.dot_general(t, w1_ref[...], (((1,), (0,)), ((), ())),
                                preferred_element_type=jnp.float32) + b1_ref[...],
            0.0)
        out_ref[...] = dinv * jax.lax.dot_general(
            z1, w2_ref[...], (((1,), (0,)), ((), ())),
            preferred_element_type=jnp.float32)

    return pl.pallas_call(
        body,
        grid=(_row_grid(N),),
        in_specs=[
            pl.BlockSpec((2, _R, D), lambda i: (0, i, 0)),
            pl.BlockSpec((_R, D), lambda i: (i, 0)),
            pl.BlockSpec((_R, 1), lambda i: (i, 0)),
            pl.BlockSpec((D, H), lambda i: (0, 0)),
            pl.BlockSpec((1, H), lambda i: (0, 0)),
            pl.BlockSpec((H, D), lambda i: (0, 0)),
        ],
        out_specs=pl.BlockSpec((_R, D), lambda i: (i, 0)),
        out_shape=jax.ShapeDtypeStruct((N, D), jnp.float32),
    )


@functools.lru_cache(maxsize=None)
def _make_stage_b(N, D):
    def body(u_ref, ms_ref, dinv_ref, b2_ref, out_ref):
        dinv = dinv_ref[...]
        z2 = jnp.maximum(dinv * (u_ref[0] + u_ref[1] + ms_ref[...]) + b2_ref[...], 0.0)
        out_ref[...] = dinv * z2

    return pl.pallas_call(
        body,
        grid=(_row_grid(N),),
        in_specs=[
            pl.BlockSpec((2, _R, D), lambda i: (0, i, 0)),
            pl.BlockSpec((_R, D), lambda i: (i, 0)),
            pl.BlockSpec((_R, 1), lambda i: (i, 0)),
            pl.BlockSpec((1, D), lambda i: (0, 0)),
        ],
        out_specs=pl.BlockSpec((_R, D), lambda i: (i, 0)),
        out_shape=jax.ShapeDtypeStruct((N, D), jnp.float32),
    )


@functools.lru_cache(maxsize=None)
def _make_stage_c(N, L2, H, D):
    def body(u_ref, zs_ref, dinv_ref, w3_ref, b3_ref, w5_ref, b5_ref,
             as_ref, s_ref):
        dinv = dinv_ref[...]
        q = dinv * (u_ref[0] + u_ref[1] + zs_ref[...])
        a = jnp.maximum(
            jax.lax.dot_general(q, w3_ref[...], (((1,), (0,)), ((), ())),
                                preferred_element_type=jnp.float32) + b3_ref[...],
            0.0)
        as_ref[...] = dinv * a
        s_ref[...] = jnp.maximum(
            jax.lax.dot_general(q, w5_ref[...], (((1,), (0,)), ((), ())),
                                preferred_element_type=jnp.float32) + b5_ref[...],
            0.0)

    return pl.pallas_call(
        body,
        grid=(_row_grid(N),),
        in_specs=[
            pl.BlockSpec((2, _R, D), lambda i: (0, i, 0)),
            pl.BlockSpec((_R, D), lambda i: (i, 0)),
            pl.BlockSpec((_R, 1), lambda i: (i, 0)),
            pl.BlockSpec((D, H), lambda i: (0, 0)),
            pl.BlockSpec((1, H), lambda i: (0, 0)),
            pl.BlockSpec((D, L2), lambda i: (0, 0)),
            pl.BlockSpec((1, L2), lambda i: (0, 0)),
        ],
        out_specs=[
            pl.BlockSpec((_R, H), lambda i: (i, 0)),
            pl.BlockSpec((_R, L2), lambda i: (i, 0)),
        ],
        out_shape=[
            jax.ShapeDtypeStruct((N, H), jnp.float32),
            jax.ShapeDtypeStruct((N, L2), jnp.float32),
        ],
    )


@functools.lru_cache(maxsize=None)
def _make_stage_d(N, H, D):
    def body(u_ref, as_ref, dinv_ref, w4_ref, b4_ref, out_ref):
        dinv = dinv_ref[...]
        t = dinv * (u_ref[0] + u_ref[1] + as_ref[...])
        out_ref[...] = jnp.maximum(
            jax.lax.dot_general(t, w4_ref[...], (((1,), (0,)), ((), ())),
                                preferred_element_type=jnp.float32) + b4_ref[...],
            0.0)

    return pl.pallas_call(
        body,
        grid=(_row_grid(N),),
        in_specs=[
            pl.BlockSpec((2, _R, H), lambda i: (0, i, 0)),
            pl.BlockSpec((_R, H), lambda i: (i, 0)),
            pl.BlockSpec((_R, 1), lambda i: (i, 0)),
            pl.BlockSpec((H, D), lambda i: (0, 0)),
            pl.BlockSpec((1, D), lambda i: (0, 0)),
        ],
        out_specs=pl.BlockSpec((_R, D), lambda i: (i, 0)),
        out_shape=jax.ShapeDtypeStruct((N, D), jnp.float32),
    )


@functools.lru_cache(maxsize=None)
def _make_adj(N, L2):
    BR, BC = 512, 1024

    def body(sr_ref, sc_ref, out_ref):
        out_ref[...] = jax.lax.dot_general(
            sr_ref[...], sc_ref[...], (((1,), (1,)), ((), ())),
            preferred_element_type=jnp.float32)

    return pl.pallas_call(
        body,
        grid=(pl.cdiv(N, BR), pl.cdiv(N, BC)),
        in_specs=[
            pl.BlockSpec((BR, L2), lambda i, j: (i, 0)),
            pl.BlockSpec((BC, L2), lambda i, j: (j, 0)),
        ],
        out_specs=pl.BlockSpec((BR, BC), lambda i, j: (i, j)),
        out_shape=jax.ShapeDtypeStruct((N, N), jnp.float32),
    )


def kernel(x, edge_index, W1, b1, W2, b2, W3, b3, W4, b4, W5, b5):
    N, D = x.shape
    E = edge_index.shape[1]
    H = W1.shape[1]
    L2 = W2.shape[1]
    CH = 80
    NB = 5
    NW = NC * NS
    nch = E // (NW * CH)
    src = edge_index[0].reshape(NW, NB, nch // NB, CH)
    dst = edge_index[1].reshape(NW, NB, nch // NB, CH)
    dstd = edge_index[1].reshape(NW, nch, CH)
    b1r = b1.reshape(1, H)
    b2r = b2.reshape(1, L2)
    b3r = b3.reshape(1, H)
    b4r = b4.reshape(1, D)
    b5r = b5.reshape(1, L2)

    W2p = jnp.pad(W2, ((0, 0), (0, D - L2)))
    b2p = jnp.pad(b2r, ((0, 0), (0, D - L2)))
    W3p = jnp.pad(W3, ((0, D - L2), (0, 0)))
    W5p = jnp.pad(W5, ((0, D - L2), (0, 0)))

    scat = _make_scatter(N, E, D)
    degp = _make_deg(N, E)(dstd).reshape(NC, N)
    dinv, xs = _make_prescale(N, D)(
        degp[0].reshape(N, 1), degp[1].reshape(N, 1), x)
    u1 = scat(src, dst, xs)
    ms = _make_stage_a(N, D, H)(u1, xs, dinv, W1, b1r, W2p)
    u2 = scat(src, dst, ms)
    zs = _make_stage_b(N, D)(u2, ms, dinv, b2p)
    u3 = scat(src, dst, zs)
    as_, s = _make_stage_c(N, L2, H, D)(u3, zs, dinv, W3p, b3r, W5p, b5r)
    u4 = scat(src, dst, as_)
    adj = _make_adj(N, L2)(s, s)
    x_recon = _make_stage_d(N, H, D)(u4, as_, dinv, W4, b4r)
    return (x_recon, adj)
